# baseline (speedup 1.0000x reference)
import functools

import jax
import jax.numpy as jnp
from jax import lax
from jax.experimental import pallas as pl
from jax.experimental.pallas import tpu as pltpu
from jax.experimental.pallas import tpu_sc as plsc

N = 10000
NP = 10240
F_IN = 128
C = 64
G = 64
NC = 2
NS = 16
NW = NC * NS
RPT = NP // NS
CHUNK = 128
NCH = 80
E_PAD = NW * NCH * CHUNK

_mesh = plsc.VectorSubcoreMesh(core_axis_name="c", subcore_axis_name="s")
_sc_params = pltpu.CompilerParams(use_tc_tiling_on_sc=False)


@functools.partial(
    pl.kernel,
    mesh=_mesh,
    out_type=jax.ShapeDtypeStruct((NC * NP, 16), jnp.float32),
    compiler_params=_sc_params,
    scratch_types=[
        pltpu.VMEM((NCH, CHUNK), jnp.int32),
        pltpu.VMEM((CHUNK, 16), jnp.float32),
        pltpu.VMEM_SHARED((NP, 16), jnp.float32),
    ],
)
def _deg_kernel(dst_hbm, ones_hbm, zeros_hbm, out_hbm, dst_v, ones_v, hist):
    c = lax.axis_index("c")
    s = lax.axis_index("s")
    wid = s * NC + c
    pltpu.sync_copy(dst_hbm.at[wid], dst_v)
    pltpu.sync_copy(ones_hbm, ones_v)
    pltpu.sync_copy(zeros_hbm, hist.at[pl.ds(s * RPT, RPT)])
    plsc.subcore_barrier()

    def body(j, carry):
        pltpu.sync_copy(ones_v, hist.at[dst_v.at[j]], add=True)
        return carry

    lax.fori_loop(0, NCH, body, 0)
    plsc.subcore_barrier()
    pltpu.sync_copy(hist.at[pl.ds(s * RPT, RPT)],
                    out_hbm.at[pl.ds(c * NP + s * RPT, RPT)])


GRP = 4
NGRP = NCH // GRP
NCHI = NCH + GRP


@functools.partial(
    pl.kernel,
    mesh=_mesh,
    out_type=jax.ShapeDtypeStruct((NC * NP, C), jnp.float32),
    compiler_params=_sc_params,
    scratch_types=[
        pltpu.VMEM((NCHI, CHUNK), jnp.int32),
        pltpu.VMEM((NCH, CHUNK), jnp.int32),
        pltpu.VMEM((GRP * CHUNK, C), jnp.float32),
        pltpu.VMEM((GRP * CHUNK, C), jnp.float32),
        pltpu.VMEM_SHARED((NP, C), jnp.float32),
        pltpu.SemaphoreType.DMA,
        pltpu.SemaphoreType.DMA,
        pltpu.SemaphoreType.DMA,
        pltpu.SemaphoreType.DMA,
    ],
)
def _scatter_kernel(hs_hbm, src_hbm, dst_hbm, zeros_hbm, out_hbm,
                    src_v, dst_v, buf_a, buf_b, acc,
                    sga, sgb, ssa, ssb):
    c = lax.axis_index("c")
    s = lax.axis_index("s")
    wid = s * NC + c
    pltpu.sync_copy(src_hbm.at[wid], src_v)
    pltpu.sync_copy(dst_hbm.at[wid], dst_v)
    pltpu.sync_copy(zeros_hbm, acc.at[pl.ds(s * RPT, RPT)])
    plsc.subcore_barrier()

    def start_g(buf, sem, g):
        for b in range(GRP):
            pltpu.make_async_copy(
                hs_hbm.at[src_v.at[g * GRP + b]],
                buf.at[pl.ds(b * CHUNK, CHUNK)], sem).start()

    def wait_g(buf, sem):
        for b in range(GRP):
            pltpu.make_async_copy(
                hs_hbm.at[src_v.at[b]],
                buf.at[pl.ds(b * CHUNK, CHUNK)], sem).wait()

    def scat(buf, sem, g):
        for b in range(GRP):
            pltpu.make_async_copy(
                buf.at[pl.ds(b * CHUNK, CHUNK)],
                acc.at[dst_v.at[g * GRP + b]], sem).start(add=True)
        for b in range(GRP):
            pltpu.make_async_copy(
                buf.at[pl.ds(b * CHUNK, CHUNK)],
                acc.at[dst_v.at[g * GRP + b]], sem).wait()

    start_g(buf_a, sga, 0)

    def body(i, carry):
        g0 = 2 * i
        start_g(buf_b, sgb, g0 + 1)
        wait_g(buf_a, sga)
        scat(buf_a, ssa, g0)
        start_g(buf_a, sga, g0 + 2)
        wait_g(buf_b, sgb)
        scat(buf_b, ssb, g0 + 1)
        return carry

    lax.fori_loop(0, NGRP // 2, body, 0)
    wait_g(buf_a, sga)
    plsc.subcore_barrier()
    pltpu.sync_copy(acc.at[pl.ds(s * RPT, RPT)],
                    out_hbm.at[pl.ds(c * NP + s * RPT, RPT)])


def _dis(hist_ref):
    deg = hist_ref[0, :, 0:1] + hist_ref[1, :, 0:1] + 1.0
    return lax.rsqrt(deg)


def _layer1_body(x_ref, hist_ref, w_ref, o_ref):
    h = jnp.dot(x_ref[...], w_ref[...], preferred_element_type=jnp.float32)
    o_ref[...] = h * _dis(hist_ref)


def _mid_body(t_ref, hsp_ref, hist_ref, b_ref, w_ref, o_ref):
    dis = _dis(hist_ref)
    agg = dis * (t_ref[0] + t_ref[1] + hsp_ref[...]) + b_ref[...]
    h = jnp.maximum(agg, 0.0)
    o_ref[...] = jnp.dot(h, w_ref[...], preferred_element_type=jnp.float32) * dis


def _final_body(t_ref, hs3_ref, hist_ref, b3_ref, seg_ref,
                wl1_ref, bl1_ref, wl2_ref, bl2_ref, o_ref, p_acc):
    i = pl.program_id(0)

    @pl.when(i == 0)
    def _():
        p_acc[...] = jnp.zeros_like(p_acc)

    dis = _dis(hist_ref)
    h3 = dis * (t_ref[0] + t_ref[1] + hs3_ref[...]) + b3_ref[...]
    onehot = (seg_ref[...] == lax.broadcasted_iota(jnp.int32, (RPT, G), 1)
              ).astype(jnp.float32)
    p_acc[...] += lax.dot_general(onehot, h3, (((0,), (0,)), ((), ())),
                                  preferred_element_type=jnp.float32)

    @pl.when(i == pl.num_programs(0) - 1)
    def _():
        p = p_acc[...]
        pr = jnp.maximum(
            jnp.dot(p, wl1_ref[...], preferred_element_type=jnp.float32)
            + bl1_ref[...], 0.0)
        o_ref[...] = (jnp.dot(pr, wl2_ref[...],
                              preferred_element_type=jnp.float32)
                      + bl2_ref[...])


def _layer1(xp, hist, W1):
    return pl.pallas_call(
        _layer1_body,
        grid=(NS,),
        in_specs=[
            pl.BlockSpec((RPT, F_IN), lambda i: (i, 0)),
            pl.BlockSpec((2, RPT, 16), lambda i: (0, i, 0)),
            pl.BlockSpec((F_IN, C), lambda i: (0, 0)),
        ],
        out_specs=pl.BlockSpec((RPT, C), lambda i: (i, 0)),
        out_shape=jax.ShapeDtypeStruct((NP, C), jnp.float32),
    )(xp, hist, W1)


def _mid(t, hsp, hist, bias, W):
    return pl.pallas_call(
        _mid_body,
        grid=(NS,),
        in_specs=[
            pl.BlockSpec((2, RPT, C), lambda i: (0, i, 0)),
            pl.BlockSpec((RPT, C), lambda i: (i, 0)),
            pl.BlockSpec((2, RPT, 16), lambda i: (0, i, 0)),
            pl.BlockSpec((1, C), lambda i: (0, 0)),
            pl.BlockSpec((C, C), lambda i: (0, 0)),
        ],
        out_specs=pl.BlockSpec((RPT, C), lambda i: (i, 0)),
        out_shape=jax.ShapeDtypeStruct((NP, C), jnp.float32),
    )(t, hsp, hist, bias, W)


def _final(t, hs3, hist, b3, segp, Wl1, bl1, Wl2, bl2):
    return pl.pallas_call(
        _final_body,
        grid=(NS,),
        in_specs=[
            pl.BlockSpec((2, RPT, C), lambda i: (0, i, 0)),
            pl.BlockSpec((RPT, C), lambda i: (i, 0)),
            pl.BlockSpec((2, RPT, 16), lambda i: (0, i, 0)),
            pl.BlockSpec((1, C), lambda i: (0, 0)),
            pl.BlockSpec((RPT, 1), lambda i: (i, 0)),
            pl.BlockSpec((C, 32), lambda i: (0, 0)),
            pl.BlockSpec((1, 32), lambda i: (0, 0)),
            pl.BlockSpec((32, 1), lambda i: (0, 0)),
            pl.BlockSpec((1, 1), lambda i: (0, 0)),
        ],
        out_specs=pl.BlockSpec((G, 1), lambda i: (0, 0)),
        out_shape=jax.ShapeDtypeStruct((G, 1), jnp.float32),
        scratch_shapes=[pltpu.VMEM((G, C), jnp.float32)],
    )(t, hs3, hist, b3, segp, Wl1, bl1, Wl2, bl2)


def kernel(x, e, b, W1, b1, W2, b2, W3, b3, Wl1, bl1, Wl2, bl2):
    E = e.shape[1]
    xp = jnp.pad(x, ((0, NP - N), (0, 0)))
    pad = jnp.full((E_PAD - E,), N, jnp.int32)
    srcp = jnp.concatenate([e[0], pad]).reshape(NW, NCH, CHUNK)
    srcp = jnp.concatenate(
        [srcp, jnp.full((NW, NCHI - NCH, CHUNK), N, jnp.int32)], axis=1)
    dstp = jnp.concatenate([e[1], pad]).reshape(NW, NCH, CHUNK)
    segp = jnp.concatenate([b, jnp.full((NP - N,), G, jnp.int32)]
                           ).reshape(NP, 1)
    ones16 = jnp.ones((CHUNK, 16), jnp.float32)
    zer16 = jnp.zeros((RPT, 16), jnp.float32)
    zer64 = jnp.zeros((RPT, C), jnp.float32)

    hist = _deg_kernel(dstp, ones16, zer16).reshape(2, NP, 16)
    hs1 = _layer1(xp, hist, W1)
    t1 = _scatter_kernel(hs1, srcp, dstp, zer64).reshape(2, NP, C)
    hs2 = _mid(t1, hs1, hist, b1.reshape(1, C), W2)
    t2 = _scatter_kernel(hs2, srcp, dstp, zer64).reshape(2, NP, C)
    hs3 = _mid(t2, hs2, hist, b2.reshape(1, C), W3)
    t3 = _scatter_kernel(hs3, srcp, dstp, zer64).reshape(2, NP, C)
    return _final(t3, hs3, hist, b3.reshape(1, C), segp,
                  Wl1, bl1.reshape(1, 32), Wl2, bl2.reshape(1, 1))


# speedup vs baseline: 1.4753x; 1.4753x over previous
import functools

import jax
import jax.numpy as jnp
from jax import lax
from jax.experimental import pallas as pl
from jax.experimental.pallas import tpu as pltpu
from jax.experimental.pallas import tpu_sc as plsc

N = 10000
NP = 10240
F_IN = 128
C = 64
G = 64
NC = 2
NS = 16
NW = NC * NS
RPT = NP // NS
CHUNK = 128
NCH = 80
E_PAD = NW * NCH * CHUNK

_mesh = plsc.VectorSubcoreMesh(core_axis_name="c", subcore_axis_name="s")
_sc_params = pltpu.CompilerParams(use_tc_tiling_on_sc=False)


@functools.partial(
    pl.kernel,
    mesh=_mesh,
    out_type=jax.ShapeDtypeStruct((NC * NP, 16), jnp.float32),
    compiler_params=_sc_params,
    scratch_types=[
        pltpu.VMEM((NCH, CHUNK), jnp.int32),
        pltpu.VMEM((CHUNK, 16), jnp.float32),
        pltpu.VMEM_SHARED((NP, 16), jnp.float32),
    ],
)
def _deg_kernel(dst_hbm, ones_hbm, zeros_hbm, out_hbm, dst_v, ones_v, hist):
    c = lax.axis_index("c")
    s = lax.axis_index("s")
    wid = s * NC + c
    pltpu.sync_copy(dst_hbm.at[wid], dst_v)
    pltpu.sync_copy(ones_hbm, ones_v)
    pltpu.sync_copy(zeros_hbm, hist.at[pl.ds(s * RPT, RPT)])
    plsc.subcore_barrier()

    def body(j, carry):
        pltpu.sync_copy(ones_v, hist.at[dst_v.at[j]], add=True)
        return carry

    lax.fori_loop(0, NCH, body, 0)
    plsc.subcore_barrier()
    pltpu.sync_copy(hist.at[pl.ds(s * RPT, RPT)],
                    out_hbm.at[pl.ds(c * NP + s * RPT, RPT)])


GRP = 4
NGRP = NCH // GRP
NCHI = NCH + GRP


@functools.partial(
    pl.kernel,
    mesh=_mesh,
    out_type=jax.ShapeDtypeStruct((NC * NP, C), jnp.float32),
    compiler_params=_sc_params,
    scratch_types=[
        pltpu.VMEM((NCHI, CHUNK), jnp.int32),
        pltpu.VMEM((NCH, CHUNK), jnp.int32),
        pltpu.VMEM((GRP * CHUNK, C), jnp.float32),
        pltpu.VMEM((GRP * CHUNK, C), jnp.float32),
        pltpu.VMEM_SHARED((NP, C), jnp.float32),
        pltpu.SemaphoreType.DMA,
        pltpu.SemaphoreType.DMA,
        pltpu.SemaphoreType.DMA,
        pltpu.SemaphoreType.DMA,
    ],
)
def _scatter_kernel(hs_hbm, src_hbm, dst_hbm, zeros_hbm, out_hbm,
                    src_v, dst_v, buf_a, buf_b, acc,
                    sga, sgb, ssa, ssb):
    c = lax.axis_index("c")
    s = lax.axis_index("s")
    wid = s * NC + c
    pltpu.sync_copy(src_hbm.at[wid], src_v)
    pltpu.sync_copy(dst_hbm.at[wid], dst_v)
    pltpu.sync_copy(zeros_hbm, acc.at[pl.ds(s * RPT, RPT)])
    plsc.subcore_barrier()

    def start_g(buf, sem, j):
        pltpu.make_async_copy(hs_hbm.at[src_v.at[j]],
                              buf.at[pl.ds(0, CHUNK)], sem).start()

    def wait_g(buf, sem):
        pltpu.make_async_copy(hs_hbm.at[src_v.at[0]],
                              buf.at[pl.ds(0, CHUNK)], sem).wait()

    def scat(buf, j):
        pltpu.sync_copy(buf.at[pl.ds(0, CHUNK)], acc.at[dst_v.at[j]],
                        add=True)

    start_g(buf_a, sga, 0)

    def body(i, carry):
        j = 2 * i
        wait_g(buf_a, sga)
        start_g(buf_b, sgb, j + 1)
        scat(buf_a, j)
        wait_g(buf_b, sgb)
        start_g(buf_a, sga, j + 2)
        scat(buf_b, j + 1)
        return carry

    lax.fori_loop(0, NCH // 2, body, 0)
    wait_g(buf_a, sga)
    plsc.subcore_barrier()
    pltpu.sync_copy(acc.at[pl.ds(s * RPT, RPT)],
                    out_hbm.at[pl.ds(c * NP + s * RPT, RPT)])


def _dis(hist_ref):
    deg = hist_ref[0, :, 0:1] + hist_ref[1, :, 0:1] + 1.0
    return lax.rsqrt(deg)


def _layer1_body(x_ref, hist_ref, w_ref, o_ref):
    h = jnp.dot(x_ref[...], w_ref[...], preferred_element_type=jnp.float32)
    o_ref[...] = h * _dis(hist_ref)


def _mid_body(t_ref, hsp_ref, hist_ref, b_ref, w_ref, o_ref):
    dis = _dis(hist_ref)
    agg = dis * (t_ref[0] + t_ref[1] + hsp_ref[...]) + b_ref[...]
    h = jnp.maximum(agg, 0.0)
    o_ref[...] = jnp.dot(h, w_ref[...], preferred_element_type=jnp.float32) * dis


def _final_body(t_ref, hs3_ref, hist_ref, b3_ref, seg_ref,
                wl1_ref, bl1_ref, wl2_ref, bl2_ref, o_ref, p_acc):
    i = pl.program_id(0)

    @pl.when(i == 0)
    def _():
        p_acc[...] = jnp.zeros_like(p_acc)

    dis = _dis(hist_ref)
    h3 = dis * (t_ref[0] + t_ref[1] + hs3_ref[...]) + b3_ref[...]
    onehot = (seg_ref[...] == lax.broadcasted_iota(jnp.int32, (RPT, G), 1)
              ).astype(jnp.float32)
    p_acc[...] += lax.dot_general(onehot, h3, (((0,), (0,)), ((), ())),
                                  preferred_element_type=jnp.float32)

    @pl.when(i == pl.num_programs(0) - 1)
    def _():
        p = p_acc[...]
        pr = jnp.maximum(
            jnp.dot(p, wl1_ref[...], preferred_element_type=jnp.float32)
            + bl1_ref[...], 0.0)
        o_ref[...] = (jnp.dot(pr, wl2_ref[...],
                              preferred_element_type=jnp.float32)
                      + bl2_ref[...])


def _layer1(xp, hist, W1):
    return pl.pallas_call(
        _layer1_body,
        grid=(NS,),
        in_specs=[
            pl.BlockSpec((RPT, F_IN), lambda i: (i, 0)),
            pl.BlockSpec((2, RPT, 16), lambda i: (0, i, 0)),
            pl.BlockSpec((F_IN, C), lambda i: (0, 0)),
        ],
        out_specs=pl.BlockSpec((RPT, C), lambda i: (i, 0)),
        out_shape=jax.ShapeDtypeStruct((NP, C), jnp.float32),
    )(xp, hist, W1)


def _mid(t, hsp, hist, bias, W):
    return pl.pallas_call(
        _mid_body,
        grid=(NS,),
        in_specs=[
            pl.BlockSpec((2, RPT, C), lambda i: (0, i, 0)),
            pl.BlockSpec((RPT, C), lambda i: (i, 0)),
            pl.BlockSpec((2, RPT, 16), lambda i: (0, i, 0)),
            pl.BlockSpec((1, C), lambda i: (0, 0)),
            pl.BlockSpec((C, C), lambda i: (0, 0)),
        ],
        out_specs=pl.BlockSpec((RPT, C), lambda i: (i, 0)),
        out_shape=jax.ShapeDtypeStruct((NP, C), jnp.float32),
    )(t, hsp, hist, bias, W)


def _final(t, hs3, hist, b3, segp, Wl1, bl1, Wl2, bl2):
    return pl.pallas_call(
        _final_body,
        grid=(NS,),
        in_specs=[
            pl.BlockSpec((2, RPT, C), lambda i: (0, i, 0)),
            pl.BlockSpec((RPT, C), lambda i: (i, 0)),
            pl.BlockSpec((2, RPT, 16), lambda i: (0, i, 0)),
            pl.BlockSpec((1, C), lambda i: (0, 0)),
            pl.BlockSpec((RPT, 1), lambda i: (i, 0)),
            pl.BlockSpec((C, 32), lambda i: (0, 0)),
            pl.BlockSpec((1, 32), lambda i: (0, 0)),
            pl.BlockSpec((32, 1), lambda i: (0, 0)),
            pl.BlockSpec((1, 1), lambda i: (0, 0)),
        ],
        out_specs=pl.BlockSpec((G, 1), lambda i: (0, 0)),
        out_shape=jax.ShapeDtypeStruct((G, 1), jnp.float32),
        scratch_shapes=[pltpu.VMEM((G, C), jnp.float32)],
    )(t, hs3, hist, b3, segp, Wl1, bl1, Wl2, bl2)


def kernel(x, e, b, W1, b1, W2, b2, W3, b3, Wl1, bl1, Wl2, bl2):
    E = e.shape[1]
    xp = jnp.pad(x, ((0, NP - N), (0, 0)))
    pad = jnp.full((E_PAD - E,), N, jnp.int32)
    srcp = jnp.concatenate([e[0], pad]).reshape(NW, NCH, CHUNK)
    srcp = jnp.concatenate(
        [srcp, jnp.full((NW, NCHI - NCH, CHUNK), N, jnp.int32)], axis=1)
    dstp = jnp.concatenate([e[1], pad]).reshape(NW, NCH, CHUNK)
    segp = jnp.concatenate([b, jnp.full((NP - N,), G, jnp.int32)]
                           ).reshape(NP, 1)
    ones16 = jnp.ones((CHUNK, 16), jnp.float32)
    zer16 = jnp.zeros((RPT, 16), jnp.float32)
    zer64 = jnp.zeros((RPT, C), jnp.float32)

    hist = _deg_kernel(dstp, ones16, zer16).reshape(2, NP, 16)
    hs1 = _layer1(xp, hist, W1)
    t1 = _scatter_kernel(hs1, srcp, dstp, zer64).reshape(2, NP, C)
    hs2 = _mid(t1, hs1, hist, b1.reshape(1, C), W2)
    t2 = _scatter_kernel(hs2, srcp, dstp, zer64).reshape(2, NP, C)
    hs3 = _mid(t2, hs2, hist, b2.reshape(1, C), W3)
    t3 = _scatter_kernel(hs3, srcp, dstp, zer64).reshape(2, NP, C)
    return _final(t3, hs3, hist, b3.reshape(1, C), segp,
                  Wl1, bl1.reshape(1, 32), Wl2, bl2.reshape(1, 1))


# speedup vs baseline: 3.8633x; 2.6187x over previous
import functools

import jax
import jax.numpy as jnp
from jax import lax
from jax.experimental import pallas as pl
from jax.experimental.pallas import tpu as pltpu
from jax.experimental.pallas import tpu_sc as plsc

N = 10000
NP = 10240
F_IN = 128
C = 64
G = 64
NC = 2
NS = 16
NW = NC * NS
RPT = NP // NS
CHUNK = 128
NCH = 80
E_PAD = NW * NCH * CHUNK

_mesh = plsc.VectorSubcoreMesh(core_axis_name="c", subcore_axis_name="s")
_sc_params = pltpu.CompilerParams(use_tc_tiling_on_sc=False)


@functools.partial(
    pl.kernel,
    mesh=_mesh,
    out_type=jax.ShapeDtypeStruct((NC * NP, 16), jnp.float32),
    compiler_params=_sc_params,
    scratch_types=[
        pltpu.VMEM((NCH, CHUNK), jnp.int32),
        pltpu.VMEM((CHUNK, 16), jnp.float32),
        pltpu.VMEM_SHARED((NP, 16), jnp.float32),
    ],
)
def _deg_kernel(dst_hbm, ones_hbm, zeros_hbm, out_hbm, dst_v, ones_v, hist):
    c = lax.axis_index("c")
    s = lax.axis_index("s")
    wid = s * NC + c
    pltpu.sync_copy(dst_hbm.at[wid], dst_v)
    pltpu.sync_copy(ones_hbm, ones_v)
    pltpu.sync_copy(zeros_hbm, hist.at[pl.ds(s * RPT, RPT)])
    plsc.subcore_barrier()

    def body(j, carry):
        pltpu.sync_copy(ones_v, hist.at[dst_v.at[j]], add=True)
        return carry

    lax.fori_loop(0, NCH, body, 0)
    plsc.subcore_barrier()
    pltpu.sync_copy(hist.at[pl.ds(s * RPT, RPT)],
                    out_hbm.at[pl.ds(c * NP + s * RPT, RPT)])


GRP = 4
NGRP = NCH // GRP
NCHI = NCH + GRP


@functools.partial(
    pl.kernel,
    mesh=_mesh,
    out_type=jax.ShapeDtypeStruct((NC * NP, C), jnp.float32),
    compiler_params=_sc_params,
    scratch_types=[
        pltpu.VMEM((NCHI, CHUNK), jnp.int32),
        pltpu.VMEM((NCH, CHUNK), jnp.int32),
        pltpu.VMEM((CHUNK, C), jnp.float32),
        pltpu.VMEM_SHARED((NP, C), jnp.float32),
        pltpu.VMEM_SHARED((NP, C), jnp.float32),
        pltpu.SemaphoreType.DMA,
    ],
)
def _scatter_kernel(hs_hbm, src_hbm, dst_hbm, zeros_hbm, out_hbm,
                    src_v, dst_v, buf, hs_s, acc, sem):
    c = lax.axis_index("c")
    s = lax.axis_index("s")
    wid = s * NC + c
    pltpu.sync_copy(src_hbm.at[wid], src_v)
    pltpu.sync_copy(dst_hbm.at[wid], dst_v)
    pltpu.sync_copy(hs_hbm.at[pl.ds(s * RPT, RPT)],
                    hs_s.at[pl.ds(s * RPT, RPT)])
    pltpu.sync_copy(zeros_hbm, acc.at[pl.ds(s * RPT, RPT)])
    plsc.subcore_barrier()

    def body(j, carry):
        pltpu.async_copy(hs_s.at[src_v.at[j]], buf, sem).wait()
        pltpu.sync_copy(buf, acc.at[dst_v.at[j]], add=True)
        return carry

    lax.fori_loop(0, NCH, body, 0)
    plsc.subcore_barrier()
    pltpu.sync_copy(acc.at[pl.ds(s * RPT, RPT)],
                    out_hbm.at[pl.ds(c * NP + s * RPT, RPT)])


def _dis(hist_ref):
    deg = hist_ref[0, :, 0:1] + hist_ref[1, :, 0:1] + 1.0
    return lax.rsqrt(deg)


def _layer1_body(x_ref, hist_ref, w_ref, o_ref):
    h = jnp.dot(x_ref[...], w_ref[...], preferred_element_type=jnp.float32)
    o_ref[...] = h * _dis(hist_ref)


def _mid_body(t_ref, hsp_ref, hist_ref, b_ref, w_ref, o_ref):
    dis = _dis(hist_ref)
    agg = dis * (t_ref[0] + t_ref[1] + hsp_ref[...]) + b_ref[...]
    h = jnp.maximum(agg, 0.0)
    o_ref[...] = jnp.dot(h, w_ref[...], preferred_element_type=jnp.float32) * dis


def _final_body(t_ref, hs3_ref, hist_ref, b3_ref, seg_ref,
                wl1_ref, bl1_ref, wl2_ref, bl2_ref, o_ref, p_acc):
    i = pl.program_id(0)

    @pl.when(i == 0)
    def _():
        p_acc[...] = jnp.zeros_like(p_acc)

    dis = _dis(hist_ref)
    h3 = dis * (t_ref[0] + t_ref[1] + hs3_ref[...]) + b3_ref[...]
    onehot = (seg_ref[...] == lax.broadcasted_iota(jnp.int32, (RPT, G), 1)
              ).astype(jnp.float32)
    p_acc[...] += lax.dot_general(onehot, h3, (((0,), (0,)), ((), ())),
                                  preferred_element_type=jnp.float32)

    @pl.when(i == pl.num_programs(0) - 1)
    def _():
        p = p_acc[...]
        pr = jnp.maximum(
            jnp.dot(p, wl1_ref[...], preferred_element_type=jnp.float32)
            + bl1_ref[...], 0.0)
        o_ref[...] = (jnp.dot(pr, wl2_ref[...],
                              preferred_element_type=jnp.float32)
                      + bl2_ref[...])


def _layer1(xp, hist, W1):
    return pl.pallas_call(
        _layer1_body,
        grid=(NS,),
        in_specs=[
            pl.BlockSpec((RPT, F_IN), lambda i: (i, 0)),
            pl.BlockSpec((2, RPT, 16), lambda i: (0, i, 0)),
            pl.BlockSpec((F_IN, C), lambda i: (0, 0)),
        ],
        out_specs=pl.BlockSpec((RPT, C), lambda i: (i, 0)),
        out_shape=jax.ShapeDtypeStruct((NP, C), jnp.float32),
    )(xp, hist, W1)


def _mid(t, hsp, hist, bias, W):
    return pl.pallas_call(
        _mid_body,
        grid=(NS,),
        in_specs=[
            pl.BlockSpec((2, RPT, C), lambda i: (0, i, 0)),
            pl.BlockSpec((RPT, C), lambda i: (i, 0)),
            pl.BlockSpec((2, RPT, 16), lambda i: (0, i, 0)),
            pl.BlockSpec((1, C), lambda i: (0, 0)),
            pl.BlockSpec((C, C), lambda i: (0, 0)),
        ],
        out_specs=pl.BlockSpec((RPT, C), lambda i: (i, 0)),
        out_shape=jax.ShapeDtypeStruct((NP, C), jnp.float32),
    )(t, hsp, hist, bias, W)


def _final(t, hs3, hist, b3, segp, Wl1, bl1, Wl2, bl2):
    return pl.pallas_call(
        _final_body,
        grid=(NS,),
        in_specs=[
            pl.BlockSpec((2, RPT, C), lambda i: (0, i, 0)),
            pl.BlockSpec((RPT, C), lambda i: (i, 0)),
            pl.BlockSpec((2, RPT, 16), lambda i: (0, i, 0)),
            pl.BlockSpec((1, C), lambda i: (0, 0)),
            pl.BlockSpec((RPT, 1), lambda i: (i, 0)),
            pl.BlockSpec((C, 32), lambda i: (0, 0)),
            pl.BlockSpec((1, 32), lambda i: (0, 0)),
            pl.BlockSpec((32, 1), lambda i: (0, 0)),
            pl.BlockSpec((1, 1), lambda i: (0, 0)),
        ],
        out_specs=pl.BlockSpec((G, 1), lambda i: (0, 0)),
        out_shape=jax.ShapeDtypeStruct((G, 1), jnp.float32),
        scratch_shapes=[pltpu.VMEM((G, C), jnp.float32)],
    )(t, hs3, hist, b3, segp, Wl1, bl1, Wl2, bl2)


def kernel(x, e, b, W1, b1, W2, b2, W3, b3, Wl1, bl1, Wl2, bl2):
    E = e.shape[1]
    xp = jnp.pad(x, ((0, NP - N), (0, 0)))
    pad = jnp.full((E_PAD - E,), N, jnp.int32)
    srcp = jnp.concatenate([e[0], pad]).reshape(NW, NCH, CHUNK)
    srcp = jnp.concatenate(
        [srcp, jnp.full((NW, NCHI - NCH, CHUNK), N, jnp.int32)], axis=1)
    dstp = jnp.concatenate([e[1], pad]).reshape(NW, NCH, CHUNK)
    segp = jnp.concatenate([b, jnp.full((NP - N,), G, jnp.int32)]
                           ).reshape(NP, 1)
    ones16 = jnp.ones((CHUNK, 16), jnp.float32)
    zer16 = jnp.zeros((RPT, 16), jnp.float32)
    zer64 = jnp.zeros((RPT, C), jnp.float32)

    hist = _deg_kernel(dstp, ones16, zer16).reshape(2, NP, 16)
    hs1 = _layer1(xp, hist, W1)
    t1 = _scatter_kernel(hs1, srcp, dstp, zer64).reshape(2, NP, C)
    hs2 = _mid(t1, hs1, hist, b1.reshape(1, C), W2)
    t2 = _scatter_kernel(hs2, srcp, dstp, zer64).reshape(2, NP, C)
    hs3 = _mid(t2, hs2, hist, b2.reshape(1, C), W3)
    t3 = _scatter_kernel(hs3, srcp, dstp, zer64).reshape(2, NP, C)
    return _final(t3, hs3, hist, b3.reshape(1, C), segp,
                  Wl1, bl1.reshape(1, 32), Wl2, bl2.reshape(1, 1))


# speedup vs baseline: 4.7016x; 1.2170x over previous
import functools

import jax
import jax.numpy as jnp
from jax import lax
from jax.experimental import pallas as pl
from jax.experimental.pallas import tpu as pltpu
from jax.experimental.pallas import tpu_sc as plsc

N = 10000
NP = 10240
F_IN = 128
C = 64
G = 64
NC = 2
NS = 16
NW = NC * NS
RPT = NP // NS
CHUNK = 128
NCH = 80
E_PAD = NW * NCH * CHUNK

_mesh = plsc.VectorSubcoreMesh(core_axis_name="c", subcore_axis_name="s")
_sc_params = pltpu.CompilerParams(use_tc_tiling_on_sc=False)


@functools.partial(
    pl.kernel,
    mesh=_mesh,
    out_type=jax.ShapeDtypeStruct((NC * NP, 16), jnp.float32),
    compiler_params=_sc_params,
    scratch_types=[
        pltpu.VMEM((NCH, CHUNK), jnp.int32),
        pltpu.VMEM((CHUNK, 16), jnp.float32),
        pltpu.VMEM_SHARED((NP, 16), jnp.float32),
    ],
)
def _deg_kernel(dst_hbm, ones_hbm, zeros_hbm, out_hbm, dst_v, ones_v, hist):
    c = lax.axis_index("c")
    s = lax.axis_index("s")
    wid = s * NC + c
    pltpu.sync_copy(dst_hbm.at[wid], dst_v)
    pltpu.sync_copy(ones_hbm, ones_v)
    pltpu.sync_copy(zeros_hbm, hist.at[pl.ds(s * RPT, RPT)])
    plsc.subcore_barrier()

    def body(j, carry):
        pltpu.sync_copy(ones_v, hist.at[dst_v.at[j]], add=True)
        return carry

    lax.fori_loop(0, NCH, body, 0)
    plsc.subcore_barrier()
    pltpu.sync_copy(hist.at[pl.ds(s * RPT, RPT)],
                    out_hbm.at[pl.ds(c * NP + s * RPT, RPT)])


GRP = 4
NGRP = NCH // GRP
NCHI = NCH + GRP


@functools.partial(
    pl.kernel,
    mesh=_mesh,
    out_type=jax.ShapeDtypeStruct((NC * NP, C), jnp.float32),
    compiler_params=_sc_params,
    scratch_types=[
        pltpu.VMEM((NCHI, CHUNK), jnp.int32),
        pltpu.VMEM((NCH, CHUNK), jnp.int32),
        pltpu.VMEM((CHUNK, C), jnp.float32),
        pltpu.VMEM((CHUNK, C), jnp.float32),
        pltpu.VMEM_SHARED((NP, C), jnp.float32),
        pltpu.VMEM_SHARED((NP, C), jnp.float32),
        pltpu.SemaphoreType.DMA,
        pltpu.SemaphoreType.DMA,
    ],
)
def _scatter_kernel(hs_hbm, src_hbm, dst_hbm, zeros_hbm, out_hbm,
                    src_v, dst_v, buf_a, buf_b, hs_s, acc, sga, sgb):
    c = lax.axis_index("c")
    s = lax.axis_index("s")
    wid = s * NC + c
    pltpu.sync_copy(src_hbm.at[wid], src_v)
    pltpu.sync_copy(dst_hbm.at[wid], dst_v)
    pltpu.sync_copy(hs_hbm.at[pl.ds(s * RPT, RPT)],
                    hs_s.at[pl.ds(s * RPT, RPT)])
    pltpu.sync_copy(zeros_hbm, acc.at[pl.ds(s * RPT, RPT)])
    plsc.subcore_barrier()

    def start_g(buf, sem, j):
        pltpu.make_async_copy(hs_s.at[src_v.at[j]], buf, sem).start()

    def wait_g(buf, sem):
        pltpu.make_async_copy(hs_s.at[src_v.at[0]], buf, sem).wait()

    start_g(buf_a, sga, 0)

    def body(i, carry):
        j = 2 * i
        wait_g(buf_a, sga)
        start_g(buf_b, sgb, j + 1)
        pltpu.sync_copy(buf_a, acc.at[dst_v.at[j]], add=True)
        wait_g(buf_b, sgb)
        start_g(buf_a, sga, j + 2)
        pltpu.sync_copy(buf_b, acc.at[dst_v.at[j + 1]], add=True)
        return carry

    lax.fori_loop(0, NCH // 2, body, 0)
    wait_g(buf_a, sga)
    plsc.subcore_barrier()
    pltpu.sync_copy(acc.at[pl.ds(s * RPT, RPT)],
                    out_hbm.at[pl.ds(c * NP + s * RPT, RPT)])


def _dis(hist_ref):
    deg = hist_ref[0, :, 0:1] + hist_ref[1, :, 0:1] + 1.0
    return lax.rsqrt(deg)


def _layer1_body(x_ref, hist_ref, w_ref, o_ref):
    h = jnp.dot(x_ref[...], w_ref[...], preferred_element_type=jnp.float32)
    o_ref[...] = h * _dis(hist_ref)


def _mid_body(t_ref, hsp_ref, hist_ref, b_ref, w_ref, o_ref):
    dis = _dis(hist_ref)
    agg = dis * (t_ref[0] + t_ref[1] + hsp_ref[...]) + b_ref[...]
    h = jnp.maximum(agg, 0.0)
    o_ref[...] = jnp.dot(h, w_ref[...], preferred_element_type=jnp.float32) * dis


def _final_body(t_ref, hs3_ref, hist_ref, b3_ref, seg_ref,
                wl1_ref, bl1_ref, wl2_ref, bl2_ref, o_ref, p_acc):
    i = pl.program_id(0)

    @pl.when(i == 0)
    def _():
        p_acc[...] = jnp.zeros_like(p_acc)

    dis = _dis(hist_ref)
    h3 = dis * (t_ref[0] + t_ref[1] + hs3_ref[...]) + b3_ref[...]
    onehot = (seg_ref[...] == lax.broadcasted_iota(jnp.int32, (RPT, G), 1)
              ).astype(jnp.float32)
    p_acc[...] += lax.dot_general(onehot, h3, (((0,), (0,)), ((), ())),
                                  preferred_element_type=jnp.float32)

    @pl.when(i == pl.num_programs(0) - 1)
    def _():
        p = p_acc[...]
        pr = jnp.maximum(
            jnp.dot(p, wl1_ref[...], preferred_element_type=jnp.float32)
            + bl1_ref[...], 0.0)
        o_ref[...] = (jnp.dot(pr, wl2_ref[...],
                              preferred_element_type=jnp.float32)
                      + bl2_ref[...])


def _layer1(xp, hist, W1):
    return pl.pallas_call(
        _layer1_body,
        grid=(NS,),
        in_specs=[
            pl.BlockSpec((RPT, F_IN), lambda i: (i, 0)),
            pl.BlockSpec((2, RPT, 16), lambda i: (0, i, 0)),
            pl.BlockSpec((F_IN, C), lambda i: (0, 0)),
        ],
        out_specs=pl.BlockSpec((RPT, C), lambda i: (i, 0)),
        out_shape=jax.ShapeDtypeStruct((NP, C), jnp.float32),
    )(xp, hist, W1)


def _mid(t, hsp, hist, bias, W):
    return pl.pallas_call(
        _mid_body,
        grid=(NS,),
        in_specs=[
            pl.BlockSpec((2, RPT, C), lambda i: (0, i, 0)),
            pl.BlockSpec((RPT, C), lambda i: (i, 0)),
            pl.BlockSpec((2, RPT, 16), lambda i: (0, i, 0)),
            pl.BlockSpec((1, C), lambda i: (0, 0)),
            pl.BlockSpec((C, C), lambda i: (0, 0)),
        ],
        out_specs=pl.BlockSpec((RPT, C), lambda i: (i, 0)),
        out_shape=jax.ShapeDtypeStruct((NP, C), jnp.float32),
    )(t, hsp, hist, bias, W)


def _final(t, hs3, hist, b3, segp, Wl1, bl1, Wl2, bl2):
    return pl.pallas_call(
        _final_body,
        grid=(NS,),
        in_specs=[
            pl.BlockSpec((2, RPT, C), lambda i: (0, i, 0)),
            pl.BlockSpec((RPT, C), lambda i: (i, 0)),
            pl.BlockSpec((2, RPT, 16), lambda i: (0, i, 0)),
            pl.BlockSpec((1, C), lambda i: (0, 0)),
            pl.BlockSpec((RPT, 1), lambda i: (i, 0)),
            pl.BlockSpec((C, 32), lambda i: (0, 0)),
            pl.BlockSpec((1, 32), lambda i: (0, 0)),
            pl.BlockSpec((32, 1), lambda i: (0, 0)),
            pl.BlockSpec((1, 1), lambda i: (0, 0)),
        ],
        out_specs=pl.BlockSpec((G, 1), lambda i: (0, 0)),
        out_shape=jax.ShapeDtypeStruct((G, 1), jnp.float32),
        scratch_shapes=[pltpu.VMEM((G, C), jnp.float32)],
    )(t, hs3, hist, b3, segp, Wl1, bl1, Wl2, bl2)


def kernel(x, e, b, W1, b1, W2, b2, W3, b3, Wl1, bl1, Wl2, bl2):
    E = e.shape[1]
    xp = jnp.pad(x, ((0, NP - N), (0, 0)))
    pad = jnp.full((E_PAD - E,), N, jnp.int32)
    srcp = jnp.concatenate([e[0], pad]).reshape(NW, NCH, CHUNK)
    srcp = jnp.concatenate(
        [srcp, jnp.full((NW, NCHI - NCH, CHUNK), N, jnp.int32)], axis=1)
    dstp = jnp.concatenate([e[1], pad]).reshape(NW, NCH, CHUNK)
    segp = jnp.concatenate([b, jnp.full((NP - N,), G, jnp.int32)]
                           ).reshape(NP, 1)
    ones16 = jnp.ones((CHUNK, 16), jnp.float32)
    zer16 = jnp.zeros((RPT, 16), jnp.float32)
    zer64 = jnp.zeros((RPT, C), jnp.float32)

    hist = _deg_kernel(dstp, ones16, zer16).reshape(2, NP, 16)
    hs1 = _layer1(xp, hist, W1)
    t1 = _scatter_kernel(hs1, srcp, dstp, zer64).reshape(2, NP, C)
    hs2 = _mid(t1, hs1, hist, b1.reshape(1, C), W2)
    t2 = _scatter_kernel(hs2, srcp, dstp, zer64).reshape(2, NP, C)
    hs3 = _mid(t2, hs2, hist, b2.reshape(1, C), W3)
    t3 = _scatter_kernel(hs3, srcp, dstp, zer64).reshape(2, NP, C)
    return _final(t3, hs3, hist, b3.reshape(1, C), segp,
                  Wl1, bl1.reshape(1, 32), Wl2, bl2.reshape(1, 1))
